# baseline (speedup 1.0000x reference)
import functools

import jax
import jax.numpy as jnp
import numpy as np
from jax.experimental import pallas as pl
from jax.experimental.pallas import tpu as pltpu

Q, D, CAP, A, K_NN, H = 1024, 128, 100000, 8, 32, 64
CAP_PAD = 102400
CHUNK = 2048
NSTEP = CAP_PAD // CHUNK


def _mlp_body(obs_ref, w1_ref, b1_ref, w2_ref, b2_ref, w3_ref, b3_ref, out_ref):
    h = jax.nn.relu(
        jax.lax.dot_general(obs_ref[...], w1_ref[...],
                            (((1,), (1,)), ((), ())),
                            preferred_element_type=jnp.float32)
        + b1_ref[...][None, :])
    h = jax.nn.relu(
        jax.lax.dot_general(h, w2_ref[...], (((1,), (1,)), ((), ())),
                            preferred_element_type=jnp.float32)
        + b2_ref[...][None, :])
    out_ref[...] = (
        jax.lax.dot_general(h, w3_ref[...], (((1,), (1,)), ((), ())),
                            preferred_element_type=jnp.float32)
        + b3_ref[...][None, :])


def _mlp(observation, W1, b1, W2, b2, W3, b3):
    return pl.pallas_call(
        _mlp_body,
        out_shape=jax.ShapeDtypeStruct((Q, A), jnp.float32),
    )(observation, W1, b1, W2, b2, W3, b3)


def _dist_body(obs_ref, obs_sq_ref, keys_ref, key_sq_ref, dists_ref, m_ref):
    i = pl.program_id(0)
    dot = jax.lax.dot_general(obs_ref[...], keys_ref[...],
                              (((1,), (1,)), ((), ())),
                              preferred_element_type=jnp.float32)
    d = (obs_sq_ref[...] - 2.0 * dot) + key_sq_ref[...][None, :]
    dists_ref[...] = d
    dmin = jnp.min(d.reshape(Q, CHUNK // 128, 128), axis=1)

    @pl.when(i == 0)
    def _():
        m_ref[...] = dmin

    @pl.when(i > 0)
    def _():
        m_ref[...] = jnp.minimum(m_ref[...], dmin)


def _dists(observation, obs_sq, keys_p, key_sq_p):
    return pl.pallas_call(
        _dist_body,
        grid=(NSTEP,),
        in_specs=[
            pl.BlockSpec((Q, D), lambda i: (0, 0)),
            pl.BlockSpec((Q, 1), lambda i: (0, 0)),
            pl.BlockSpec((CHUNK, D), lambda i: (i, 0)),
            pl.BlockSpec((CHUNK,), lambda i: (i,)),
        ],
        out_specs=[
            pl.BlockSpec((Q, CHUNK), lambda i: (0, i)),
            pl.BlockSpec((Q, 128), lambda i: (0, 0)),
        ],
        out_shape=[
            jax.ShapeDtypeStruct((Q, CAP_PAD), jnp.float32),
            jax.ShapeDtypeStruct((Q, 128), jnp.float32),
        ],
    )(observation, obs_sq, keys_p, key_sq_p)


def kernel(observation, keys, values, W1, b1, W2, b2, W3, b3):
    obs_sq = jnp.sum(observation * observation, axis=-1, keepdims=True)
    key_sq = jnp.sum(keys * keys, axis=-1)
    keys_p = jnp.pad(keys, ((0, CAP_PAD - CAP), (0, 0)))
    key_sq_p = jnp.pad(key_sq, (0, CAP_PAD - CAP),
                       constant_values=np.inf)
    q_net_q = _mlp(observation, W1, b1, W2, b2, W3, b3)
    dists, _m = _dists(observation, obs_sq, keys_p, key_sq_p)
    _, idx = jax.lax.top_k(-dists, K_NN)
    nn_vals = jnp.take(values, idx, axis=0)
    qec_q = jnp.mean(nn_vals, axis=1)
    q = qec_q + q_net_q
    return jnp.argmax(q, axis=-1)


# speedup vs baseline: 8.0793x; 8.0793x over previous
import functools

import jax
import jax.numpy as jnp
import numpy as np
from jax import lax
from jax.experimental import pallas as pl
from jax.experimental.pallas import tpu as pltpu
from jax.experimental.pallas import tpu_sc as plsc

Q, D, CAP, A, K_NN, H = 1024, 128, 100000, 8, 32, 64
CAP_PAD = 102400
CHUNK = 2048
NSTEP = CAP_PAD // CHUNK

NC, NS, L = 2, 16, 16
NW = NC * NS
QPW = Q // NW
SCHUNK = 12800
NCHUNK = CAP_PAD // SCHUNK
VPG = 16
NGRP = SCHUNK // (VPG * L)
SURV_CAP = 1040
SURV_BUF = SURV_CAP + 16


def _mlp_body(obs_ref, w1_ref, b1_ref, w2_ref, b2_ref, w3_ref, b3_ref, out_ref):
    h = jax.nn.relu(
        lax.dot_general(obs_ref[...], w1_ref[...], (((1,), (1,)), ((), ())),
                        preferred_element_type=jnp.float32)
        + b1_ref[...][None, :])
    h = jax.nn.relu(
        lax.dot_general(h, w2_ref[...], (((1,), (1,)), ((), ())),
                        preferred_element_type=jnp.float32)
        + b2_ref[...][None, :])
    out_ref[...] = (
        lax.dot_general(h, w3_ref[...], (((1,), (1,)), ((), ())),
                        preferred_element_type=jnp.float32)
        + b3_ref[...][None, :])


def _mlp(observation, W1, b1, W2, b2, W3, b3):
    return pl.pallas_call(
        _mlp_body,
        out_shape=jax.ShapeDtypeStruct((Q, A), jnp.float32),
    )(observation, W1, b1, W2, b2, W3, b3)


def _dist_body(obs_ref, obs_sq_ref, keys_ref, key_sq_ref, dists_ref, m_ref):
    i = pl.program_id(0)
    dot = lax.dot_general(obs_ref[...], keys_ref[...], (((1,), (1,)), ((), ())),
                          preferred_element_type=jnp.float32)
    d = (obs_sq_ref[...] - 2.0 * dot) + key_sq_ref[...][None, :]
    dists_ref[...] = d
    dmin = jnp.min(d.reshape(Q, CHUNK // 128, 128), axis=1)

    @pl.when(i == 0)
    def _():
        m_ref[...] = dmin

    @pl.when(i > 0)
    def _():
        m_ref[...] = jnp.minimum(m_ref[...], dmin)


def _dists(observation, obs_sq, keys_p, key_sq_p):
    return pl.pallas_call(
        _dist_body,
        grid=(NSTEP,),
        in_specs=[
            pl.BlockSpec((Q, D), lambda i: (0, 0)),
            pl.BlockSpec((Q, 1), lambda i: (0, 0)),
            pl.BlockSpec((CHUNK, D), lambda i: (i, 0)),
            pl.BlockSpec((CHUNK,), lambda i: (i,)),
        ],
        out_specs=[
            pl.BlockSpec((Q, CHUNK), lambda i: (0, i)),
            pl.BlockSpec((Q, 128), lambda i: (0, 0)),
        ],
        out_shape=[
            jax.ShapeDtypeStruct((Q, CAP_PAD), jnp.float32),
            jax.ShapeDtypeStruct((Q, 128), jnp.float32),
        ],
    )(observation, obs_sq, keys_p, key_sq_p)


_INF = np.float32(np.inf)
_NINF = np.float32(-np.inf)
_BIG = np.int32(1 << 30)


def _sc_body(dists_hbm, m_hbm, qnet_hbm, vals_hbm, out_hbm,
             mrow_v, bufa, bufb, sdist, sidx, selidx_v, sel8_v, vgath_v,
             qnet_v, act_v, sem_a, sem_b, sem_g):
    wid = lax.axis_index("s") * NC + lax.axis_index("c")
    q0 = wid * QPW
    lane = lax.iota(jnp.int32, L)
    pltpu.sync_copy(qnet_hbm.at[pl.ds(q0 * 16, QPW * 16)], qnet_v)

    @pl.loop(0, QPW)
    def _per_query(qq):
        q = q0 + qq
        pltpu.sync_copy(m_hbm.at[q], mrow_v)
        t = _NINF
        for g in range(8):
            m16 = mrow_v[pl.ds(g * L, L)]
            tg = _NINF
            for _r in range(4):
                tg = jnp.min(m16)
                m16 = jnp.where(m16 == tg, _INF, m16)
            t = jnp.maximum(t, tg)

        bufs = (bufa, bufb)
        sems = (sem_a, sem_b)
        desc = pltpu.async_copy(
            dists_hbm.at[q, pl.ds(0, SCHUNK)], bufs[0], sems[0])
        off = jnp.int32(0)
        for c in range(NCHUNK):
            nxt = None
            if c + 1 < NCHUNK:
                nxt = pltpu.async_copy(
                    dists_hbm.at[q, pl.ds((c + 1) * SCHUNK, SCHUNK)],
                    bufs[(c + 1) % 2], sems[(c + 1) % 2])
            desc.wait()
            buf = bufs[c % 2]
            cbase = c * SCHUNK

            @pl.loop(0, NGRP, init_carry=off)
            def off(grp, off):
                base = grp * (VPG * L)
                d16s, masks = [], []
                anym = None
                for v in range(VPG):
                    d16 = buf[pl.ds(base + v * L, L)]
                    mk = d16 <= t
                    d16s.append(d16)
                    masks.append(mk)
                    anym = mk if anym is None else (anym | mk)

                def slow(off):
                    for v in range(VPG):
                        mk = masks[v]
                        cum = plsc.cumsum(mk.astype(jnp.int32))
                        pos = off + cum - 1
                        gidx = lane + (cbase + base + v * L)
                        plsc.store_scatter(sdist, [pos], d16s[v], mask=mk)
                        plsc.store_scatter(sidx, [pos], gidx, mask=mk)
                        off = jnp.minimum(
                            off + jnp.sum(mk.astype(jnp.int32)),
                            jnp.int32(SURV_CAP))
                    return off

                return lax.cond(jnp.any(anym), slow, lambda o: o, off)

            desc = nxt

        offv = jnp.full((L,), off, jnp.int32)
        plsc.store_scatter(sdist, [offv + lane],
                           jnp.full((L,), _INF, jnp.float32))
        nv = (off + 15) // 16

        @pl.loop(0, K_NN)
        def _sel(k):
            @pl.loop(0, nv, init_carry=jnp.full((L,), _INF, jnp.float32))
            def dacc(v, acc):
                return jnp.minimum(acc, sdist[pl.ds(v * L, L)])
            dmin = jnp.min(dacc)

            @pl.loop(0, nv, init_carry=jnp.full((L,), _BIG, jnp.int32))
            def pacc(v, acc):
                d16 = sdist[pl.ds(v * L, L)]
                return jnp.minimum(
                    acc, jnp.where(d16 == dmin, lane + v * L, _BIG))
            p = jnp.min(pacc)
            pv = p // L
            plane = p - pv * L

            i16 = sidx[pl.ds(pv * L, L)]
            ival = jnp.max(jnp.where(lane == plane, i16, jnp.int32(-1)))
            plsc.store_scatter(selidx_v, [jnp.full((L,), k, jnp.int32)],
                               jnp.full((L,), ival, jnp.int32),
                               mask=lane == 0)
            d16 = sdist[pl.ds(pv * L, L)]
            sdist[pl.ds(pv * L, L)] = jnp.where(lane == plane, _INF, d16)

        s16a = selidx_v[pl.ds(0, L)]
        s16b = selidx_v[pl.ds(L, L)]
        for a in range(A):
            sel8_v[pl.ds(a * 32, L)] = s16a * A + a
            sel8_v[pl.ds(a * 32 + L, L)] = s16b * A + a
        descs = [
            pltpu.async_copy(vals_hbm.at[sel8_v.at[pl.ds(a * 32, 32)]],
                             vgath_v.at[pl.ds(a * 32, 32)], sem_g)
            for a in range(A)
        ]
        for dsc in descs:
            dsc.wait()

        qrow = qnet_v[pl.ds(qq * 16, L)]
        qbuf = jnp.full((L,), _NINF, jnp.float32)
        for a in range(A):
            u = (vgath_v[pl.ds(a * 32, L)] + vgath_v[pl.ds(a * 32 + L, L)])
            qa = jnp.sum(u) * jnp.float32(1.0 / K_NN)
            qa = qa + jnp.max(jnp.where(lane == a, qrow, _NINF))
            qbuf = jnp.where(lane == a, qa, qbuf)
        qm = jnp.max(qbuf)
        act = jnp.min(jnp.where(qbuf == qm, lane, _BIG))
        plsc.store_scatter(act_v, [jnp.full((L,), qq, jnp.int32)],
                           jnp.full((L,), act, jnp.int32), mask=lane == 0)

    pltpu.sync_copy(act_v.at[pl.ds(0, QPW)], out_hbm.at[pl.ds(q0, QPW)])


def _sc_select(dists, m, qnet16, values_flat):
    mesh = plsc.VectorSubcoreMesh(core_axis_name="c", subcore_axis_name="s")
    return pl.kernel(
        _sc_body,
        out_type=jax.ShapeDtypeStruct((Q,), jnp.int32),
        mesh=mesh,
        compiler_params=pltpu.CompilerParams(needs_layout_passes=False),
        scratch_types=[
            pltpu.VMEM((128,), jnp.float32),
            pltpu.VMEM((SCHUNK,), jnp.float32),
            pltpu.VMEM((SCHUNK,), jnp.float32),
            pltpu.VMEM((SURV_BUF,), jnp.float32),
            pltpu.VMEM((SURV_BUF,), jnp.int32),
            pltpu.VMEM((K_NN + 16,), jnp.int32),
            pltpu.VMEM((A * 32,), jnp.int32),
            pltpu.VMEM((A * 32,), jnp.float32),
            pltpu.VMEM((QPW * 16,), jnp.float32),
            pltpu.VMEM((QPW + 16,), jnp.int32),
            pltpu.SemaphoreType.DMA,
            pltpu.SemaphoreType.DMA,
            pltpu.SemaphoreType.DMA,
        ],
    )(dists, m, qnet16, values_flat)


def kernel(observation, keys, values, W1, b1, W2, b2, W3, b3):
    obs_sq = jnp.sum(observation * observation, axis=-1, keepdims=True)
    key_sq = jnp.sum(keys * keys, axis=-1)
    keys_p = jnp.pad(keys, ((0, CAP_PAD - CAP), (0, 0)))
    key_sq_p = jnp.pad(key_sq, (0, CAP_PAD - CAP), constant_values=np.inf)
    q_net_q = _mlp(observation, W1, b1, W2, b2, W3, b3)
    dists, m = _dists(observation, obs_sq, keys_p, key_sq_p)
    qnet16 = jnp.concatenate([q_net_q, q_net_q], axis=1).reshape(-1)
    return _sc_select(dists, m, qnet16, values.reshape(-1))


# speedup vs baseline: 8.5429x; 1.0574x over previous
import functools

import jax
import jax.numpy as jnp
import numpy as np
from jax import lax
from jax.experimental import pallas as pl
from jax.experimental.pallas import tpu as pltpu
from jax.experimental.pallas import tpu_sc as plsc

Q, D, CAP, A, K_NN, H = 1024, 128, 100000, 8, 32, 64
CAP_PAD = 102400
CHUNK = 2048
NSTEP = CAP_PAD // CHUNK

NC, NS, L = 2, 16, 16
NW = NC * NS
QPW = Q // NW
SCHUNK = 12800
NCHUNK = CAP_PAD // SCHUNK
VPG = 16
NGRP = SCHUNK // (VPG * L)
SURV_CAP = 1040
SURV_BUF = SURV_CAP + 16


def _mlp_body(obs_ref, w1_ref, b1_ref, w2_ref, b2_ref, w3_ref, b3_ref, out_ref):
    h = jax.nn.relu(
        lax.dot_general(obs_ref[...], w1_ref[...], (((1,), (1,)), ((), ())),
                        preferred_element_type=jnp.float32)
        + b1_ref[...][None, :])
    h = jax.nn.relu(
        lax.dot_general(h, w2_ref[...], (((1,), (1,)), ((), ())),
                        preferred_element_type=jnp.float32)
        + b2_ref[...][None, :])
    out_ref[...] = (
        lax.dot_general(h, w3_ref[...], (((1,), (1,)), ((), ())),
                        preferred_element_type=jnp.float32)
        + b3_ref[...][None, :])


def _mlp(observation, W1, b1, W2, b2, W3, b3):
    return pl.pallas_call(
        _mlp_body,
        out_shape=jax.ShapeDtypeStruct((Q, A), jnp.float32),
    )(observation, W1, b1, W2, b2, W3, b3)


def _dist_body(obs_ref, obs_sq_ref, keys_ref, key_sq_ref, dists_ref, m_ref):
    i = pl.program_id(0)
    dot = lax.dot_general(obs_ref[...], keys_ref[...], (((1,), (1,)), ((), ())),
                          preferred_element_type=jnp.float32)
    d = (obs_sq_ref[...] - 2.0 * dot) + key_sq_ref[...][None, :]
    dists_ref[...] = d
    dmin = jnp.min(d.reshape(Q, CHUNK // 128, 128), axis=1)

    @pl.when(i == 0)
    def _():
        m_ref[...] = dmin

    @pl.when(i > 0)
    def _():
        m_ref[...] = jnp.minimum(m_ref[...], dmin)


def _dists(observation, obs_sq, keys_p, key_sq_p):
    return pl.pallas_call(
        _dist_body,
        grid=(NSTEP,),
        in_specs=[
            pl.BlockSpec((Q, D), lambda i: (0, 0)),
            pl.BlockSpec((Q, 1), lambda i: (0, 0)),
            pl.BlockSpec((CHUNK, D), lambda i: (i, 0)),
            pl.BlockSpec((CHUNK,), lambda i: (i,)),
        ],
        out_specs=[
            pl.BlockSpec((Q, CHUNK), lambda i: (0, i)),
            pl.BlockSpec((Q, 128), lambda i: (0, 0)),
        ],
        out_shape=[
            jax.ShapeDtypeStruct((Q, CAP_PAD), jnp.float32),
            jax.ShapeDtypeStruct((Q, 128), jnp.float32),
        ],
    )(observation, obs_sq, keys_p, key_sq_p)


_INF = np.float32(np.inf)
_NINF = np.float32(-np.inf)
_BIG = np.int32(1 << 30)


def _sc_body(dists_hbm, m_hbm, qnet_hbm, vals_hbm, out_hbm,
             mrow_v, bufa, bufb, sdist, sidx, selidx_v, sel8_v, vgath_v,
             qnet_v, act_v, sem_a, sem_b, sem_g):
    wid = lax.axis_index("s") * NC + lax.axis_index("c")
    q0 = wid * QPW
    lane = lax.iota(jnp.int32, L)
    pltpu.sync_copy(qnet_hbm.at[pl.ds(q0 * 16, QPW * 16)], qnet_v)

    @pl.loop(0, QPW)
    def _per_query(qq):
        q = q0 + qq
        pltpu.sync_copy(m_hbm.at[q], mrow_v)
        t = _NINF
        for g in range(8):
            m16 = mrow_v[pl.ds(g * L, L)]
            tg = _NINF
            for _r in range(4):
                tg = jnp.min(m16)
                m16 = jnp.where(m16 == tg, _INF, m16)
            t = jnp.maximum(t, tg)

        bufs = (bufa, bufb)
        sems = (sem_a, sem_b)
        desc = pltpu.async_copy(
            dists_hbm.at[q, pl.ds(0, SCHUNK)], bufs[0], sems[0])
        off = jnp.int32(0)
        for c in range(NCHUNK):
            nxt = None
            if c + 1 < NCHUNK:
                nxt = pltpu.async_copy(
                    dists_hbm.at[q, pl.ds((c + 1) * SCHUNK, SCHUNK)],
                    bufs[(c + 1) % 2], sems[(c + 1) % 2])
            desc.wait()
            buf = bufs[c % 2]
            cbase = c * SCHUNK

            @pl.loop(0, NGRP, init_carry=off)
            def off(grp, off):
                base = grp * (VPG * L)
                d16s, masks = [], []
                anym = None
                for v in range(VPG):
                    d16 = buf[pl.ds(base + v * L, L)]
                    mk = d16 <= t
                    d16s.append(d16)
                    masks.append(mk)
                    anym = mk if anym is None else (anym | mk)

                def slow(off):
                    for v in range(VPG):
                        mk = masks[v]
                        cum = plsc.cumsum(mk.astype(jnp.int32))
                        pos = off + cum - 1
                        gidx = lane + (cbase + base + v * L)
                        plsc.store_scatter(sdist, [pos], d16s[v], mask=mk)
                        plsc.store_scatter(sidx, [pos], gidx, mask=mk)
                        off = jnp.minimum(
                            off + plsc.all_reduce_population_count(mk)[0],
                            jnp.int32(SURV_CAP))
                    return off

                nhit = plsc.all_reduce_population_count(anym)
                return lax.cond(nhit[0] > 0, slow, lambda o: o, off)

            desc = nxt

        offv = jnp.full((L,), off, jnp.int32)
        plsc.store_scatter(sdist, [offv + lane],
                           jnp.full((L,), _INF, jnp.float32))
        nv = (off + 15) // 16

        @pl.loop(0, K_NN)
        def _sel(k):
            @pl.loop(0, nv, init_carry=jnp.full((L,), _INF, jnp.float32))
            def dacc(v, acc):
                return jnp.minimum(acc, sdist[pl.ds(v * L, L)])
            dmin = jnp.min(dacc)

            @pl.loop(0, nv, init_carry=jnp.full((L,), _BIG, jnp.int32))
            def pacc(v, acc):
                d16 = sdist[pl.ds(v * L, L)]
                return jnp.minimum(
                    acc, jnp.where(d16 == dmin, lane + v * L, _BIG))
            p = jnp.min(pacc)
            pv = p // L
            plane = p - pv * L

            i16 = sidx[pl.ds(pv * L, L)]
            ival = jnp.max(jnp.where(lane == plane, i16, jnp.int32(-1)))
            plsc.store_scatter(selidx_v, [jnp.full((L,), k, jnp.int32)],
                               jnp.full((L,), ival, jnp.int32),
                               mask=lane == 0)
            d16 = sdist[pl.ds(pv * L, L)]
            sdist[pl.ds(pv * L, L)] = jnp.where(lane == plane, _INF, d16)

        s16a = selidx_v[pl.ds(0, L)]
        s16b = selidx_v[pl.ds(L, L)]
        for a in range(A):
            sel8_v[pl.ds(a * 32, L)] = s16a * A + a
            sel8_v[pl.ds(a * 32 + L, L)] = s16b * A + a
        descs = [
            pltpu.async_copy(vals_hbm.at[sel8_v.at[pl.ds(a * 32, 32)]],
                             vgath_v.at[pl.ds(a * 32, 32)], sem_g)
            for a in range(A)
        ]
        for dsc in descs:
            dsc.wait()

        qrow = qnet_v[pl.ds(qq * 16, L)]
        qbuf = jnp.full((L,), _NINF, jnp.float32)
        for a in range(A):
            u = (vgath_v[pl.ds(a * 32, L)] + vgath_v[pl.ds(a * 32 + L, L)])
            qa = jnp.sum(u) * jnp.float32(1.0 / K_NN)
            qa = qa + jnp.max(jnp.where(lane == a, qrow, _NINF))
            qbuf = jnp.where(lane == a, qa, qbuf)
        qm = jnp.max(qbuf)
        act = jnp.min(jnp.where(qbuf == qm, lane, _BIG))
        plsc.store_scatter(act_v, [jnp.full((L,), qq, jnp.int32)],
                           jnp.full((L,), act, jnp.int32), mask=lane == 0)

    pltpu.sync_copy(act_v.at[pl.ds(0, QPW)], out_hbm.at[pl.ds(q0, QPW)])


def _sc_select(dists, m, qnet16, values_flat):
    mesh = plsc.VectorSubcoreMesh(core_axis_name="c", subcore_axis_name="s")
    return pl.kernel(
        _sc_body,
        out_type=jax.ShapeDtypeStruct((Q,), jnp.int32),
        mesh=mesh,
        compiler_params=pltpu.CompilerParams(needs_layout_passes=False),
        scratch_types=[
            pltpu.VMEM((128,), jnp.float32),
            pltpu.VMEM((SCHUNK,), jnp.float32),
            pltpu.VMEM((SCHUNK,), jnp.float32),
            pltpu.VMEM((SURV_BUF,), jnp.float32),
            pltpu.VMEM((SURV_BUF,), jnp.int32),
            pltpu.VMEM((K_NN + 16,), jnp.int32),
            pltpu.VMEM((A * 32,), jnp.int32),
            pltpu.VMEM((A * 32,), jnp.float32),
            pltpu.VMEM((QPW * 16,), jnp.float32),
            pltpu.VMEM((QPW + 16,), jnp.int32),
            pltpu.SemaphoreType.DMA,
            pltpu.SemaphoreType.DMA,
            pltpu.SemaphoreType.DMA,
        ],
    )(dists, m, qnet16, values_flat)


def kernel(observation, keys, values, W1, b1, W2, b2, W3, b3):
    obs_sq = jnp.sum(observation * observation, axis=-1, keepdims=True)
    key_sq = jnp.sum(keys * keys, axis=-1)
    keys_p = jnp.pad(keys, ((0, CAP_PAD - CAP), (0, 0)))
    key_sq_p = jnp.pad(key_sq, (0, CAP_PAD - CAP), constant_values=np.inf)
    q_net_q = _mlp(observation, W1, b1, W2, b2, W3, b3)
    dists, m = _dists(observation, obs_sq, keys_p, key_sq_p)
    qnet16 = jnp.concatenate([q_net_q, q_net_q], axis=1).reshape(-1)
    return _sc_select(dists, m, qnet16, values.reshape(-1))


# speedup vs baseline: 9.0984x; 1.0650x over previous
import functools

import jax
import jax.numpy as jnp
import numpy as np
from jax import lax
from jax.experimental import pallas as pl
from jax.experimental.pallas import tpu as pltpu
from jax.experimental.pallas import tpu_sc as plsc

Q, D, CAP, A, K_NN, H = 1024, 128, 100000, 8, 32, 64
CAP_PAD = 102400
CHUNK = 2048
NSTEP = CAP_PAD // CHUNK

NC, NS, L = 2, 16, 16
NW = NC * NS
QPW = Q // NW
SCHUNK = 12800
NCHUNK = CAP_PAD // SCHUNK
VPG = 16
NGRP = SCHUNK // (VPG * L)
SURV_CAP = 1040
SURV_BUF = SURV_CAP + 16


def _mlp_body(obs_ref, w1_ref, b1_ref, w2_ref, b2_ref, w3_ref, b3_ref, out_ref):
    h = jax.nn.relu(
        lax.dot_general(obs_ref[...], w1_ref[...], (((1,), (1,)), ((), ())),
                        preferred_element_type=jnp.float32)
        + b1_ref[...][None, :])
    h = jax.nn.relu(
        lax.dot_general(h, w2_ref[...], (((1,), (1,)), ((), ())),
                        preferred_element_type=jnp.float32)
        + b2_ref[...][None, :])
    out_ref[...] = (
        lax.dot_general(h, w3_ref[...], (((1,), (1,)), ((), ())),
                        preferred_element_type=jnp.float32)
        + b3_ref[...][None, :])


def _mlp(observation, W1, b1, W2, b2, W3, b3):
    return pl.pallas_call(
        _mlp_body,
        out_shape=jax.ShapeDtypeStruct((Q, A), jnp.float32),
    )(observation, W1, b1, W2, b2, W3, b3)


def _dist_body(obs_ref, obs_sq_ref, keys_ref, key_sq_ref, dists_ref, m_ref):
    i = pl.program_id(0)
    dot = lax.dot_general(obs_ref[...], keys_ref[...], (((1,), (1,)), ((), ())),
                          preferred_element_type=jnp.float32)
    d = (obs_sq_ref[...] - 2.0 * dot) + key_sq_ref[...][None, :]
    dists_ref[...] = d
    dmin = jnp.min(d.reshape(Q, CHUNK // 128, 128), axis=1)

    @pl.when(i == 0)
    def _():
        m_ref[...] = dmin

    @pl.when(i > 0)
    def _():
        m_ref[...] = jnp.minimum(m_ref[...], dmin)


def _dists(observation, obs_sq, keys_p, key_sq_p):
    return pl.pallas_call(
        _dist_body,
        grid=(NSTEP,),
        in_specs=[
            pl.BlockSpec((Q, D), lambda i: (0, 0)),
            pl.BlockSpec((Q, 1), lambda i: (0, 0)),
            pl.BlockSpec((CHUNK, D), lambda i: (i, 0)),
            pl.BlockSpec((CHUNK,), lambda i: (i,)),
        ],
        out_specs=[
            pl.BlockSpec((Q, CHUNK), lambda i: (0, i)),
            pl.BlockSpec((Q, 128), lambda i: (0, 0)),
        ],
        out_shape=[
            jax.ShapeDtypeStruct((Q, CAP_PAD), jnp.float32),
            jax.ShapeDtypeStruct((Q, 128), jnp.float32),
        ],
    )(observation, obs_sq, keys_p, key_sq_p)


_INF = np.float32(np.inf)
_NINF = np.float32(-np.inf)
_BIG = np.int32(1 << 30)


def _sc_body(dists_hbm, m_hbm, qnet_hbm, vals_hbm, out_hbm,
             mrow_v, bufa, bufb, sdist, sidx, selidx_v, sel8_v, vgath_v,
             qnet_v, act_v, sem_a, sem_b, sem_g):
    wid = lax.axis_index("s") * NC + lax.axis_index("c")
    q0 = wid * QPW
    lane = lax.iota(jnp.int32, L)
    pltpu.sync_copy(qnet_hbm.at[pl.ds(q0 * 16, QPW * 16)], qnet_v)
    pltpu.sync_copy(m_hbm.at[pl.ds(q0 * 128, QPW * 128)], mrow_v)
    pltpu.async_copy(dists_hbm.at[q0, pl.ds(0, SCHUNK)], bufa, sem_a)

    @pl.loop(0, QPW)
    def _per_query(qq):
        q = q0 + qq
        t = _NINF
        for g in range(8):
            m16 = mrow_v[pl.ds(qq * 128 + g * L, L)]
            tg = _NINF
            for _r in range(4):
                tg = jnp.min(m16)
                m16 = jnp.where(m16 == tg, _INF, m16)
            t = jnp.maximum(t, tg)

        bufs = (bufa, bufb)
        sems = (sem_a, sem_b)
        off = jnp.int32(0)
        for c in range(NCHUNK):
            if c + 1 < NCHUNK:
                pltpu.async_copy(
                    dists_hbm.at[q, pl.ds((c + 1) * SCHUNK, SCHUNK)],
                    bufs[(c + 1) % 2], sems[(c + 1) % 2])
            else:
                qn = jnp.minimum(q + 1, jnp.int32(Q - 1))
                pltpu.async_copy(
                    dists_hbm.at[qn, pl.ds(0, SCHUNK)],
                    bufs[0], sems[0])
            pltpu.make_async_copy(
                dists_hbm.at[q, pl.ds(c * SCHUNK, SCHUNK)],
                bufs[c % 2], sems[c % 2]).wait()
            buf = bufs[c % 2]
            cbase = c * SCHUNK

            @pl.loop(0, NGRP, init_carry=off)
            def off(grp, off):
                base = grp * (VPG * L)
                d16s, masks = [], []
                anym = None
                for v in range(VPG):
                    d16 = buf[pl.ds(base + v * L, L)]
                    mk = d16 <= t
                    d16s.append(d16)
                    masks.append(mk)
                    anym = mk if anym is None else (anym | mk)

                def slow(off):
                    for v in range(VPG):
                        mk = masks[v]
                        cum = plsc.cumsum(mk.astype(jnp.int32))
                        pos = off + cum - 1
                        gidx = lane + (cbase + base + v * L)
                        plsc.store_scatter(sdist, [pos], d16s[v], mask=mk)
                        plsc.store_scatter(sidx, [pos], gidx, mask=mk)
                        off = jnp.minimum(
                            off + plsc.all_reduce_population_count(mk)[0],
                            jnp.int32(SURV_CAP))
                    return off

                nhit = plsc.all_reduce_population_count(anym)
                return lax.cond(nhit[0] > 0, slow, lambda o: o, off)

        offv = jnp.full((L,), off, jnp.int32)
        plsc.store_scatter(sdist, [offv + lane],
                           jnp.full((L,), _INF, jnp.float32))
        nv = (off + 15) // 16

        @pl.loop(0, K_NN)
        def _sel(k):
            @pl.loop(0, nv, init_carry=jnp.full((L,), _INF, jnp.float32))
            def dacc(v, acc):
                return jnp.minimum(acc, sdist[pl.ds(v * L, L)])
            dmin = jnp.min(dacc)

            @pl.loop(0, nv, init_carry=jnp.full((L,), _BIG, jnp.int32))
            def pacc(v, acc):
                d16 = sdist[pl.ds(v * L, L)]
                return jnp.minimum(
                    acc, jnp.where(d16 == dmin, lane + v * L, _BIG))
            p = jnp.min(pacc)
            pv = p // L
            plane = p - pv * L

            i16 = sidx[pl.ds(pv * L, L)]
            ival = jnp.max(jnp.where(lane == plane, i16, jnp.int32(-1)))
            plsc.store_scatter(selidx_v, [jnp.full((L,), k, jnp.int32)],
                               jnp.full((L,), ival, jnp.int32),
                               mask=lane == 0)
            d16 = sdist[pl.ds(pv * L, L)]
            sdist[pl.ds(pv * L, L)] = jnp.where(lane == plane, _INF, d16)

        s16a = selidx_v[pl.ds(0, L)]
        s16b = selidx_v[pl.ds(L, L)]
        for a in range(A):
            sel8_v[pl.ds(a * 32, L)] = s16a * A + a
            sel8_v[pl.ds(a * 32 + L, L)] = s16b * A + a
        descs = [
            pltpu.async_copy(vals_hbm.at[sel8_v.at[pl.ds(a * 32, 32)]],
                             vgath_v.at[pl.ds(a * 32, 32)], sem_g)
            for a in range(A)
        ]
        for dsc in descs:
            dsc.wait()

        qrow = qnet_v[pl.ds(qq * 16, L)]
        qbuf = jnp.full((L,), _NINF, jnp.float32)
        for a in range(A):
            u = (vgath_v[pl.ds(a * 32, L)] + vgath_v[pl.ds(a * 32 + L, L)])
            qa = jnp.sum(u) * jnp.float32(1.0 / K_NN)
            qa = qa + jnp.max(jnp.where(lane == a, qrow, _NINF))
            qbuf = jnp.where(lane == a, qa, qbuf)
        qm = jnp.max(qbuf)
        act = jnp.min(jnp.where(qbuf == qm, lane, _BIG))
        plsc.store_scatter(act_v, [jnp.full((L,), qq, jnp.int32)],
                           jnp.full((L,), act, jnp.int32), mask=lane == 0)

    pltpu.make_async_copy(
        dists_hbm.at[q0, pl.ds(0, SCHUNK)], bufa, sem_a).wait()
    pltpu.sync_copy(act_v.at[pl.ds(0, QPW)], out_hbm.at[pl.ds(q0, QPW)])


def _sc_select(dists, m, qnet16, values_flat):
    mesh = plsc.VectorSubcoreMesh(core_axis_name="c", subcore_axis_name="s")
    return pl.kernel(
        _sc_body,
        out_type=jax.ShapeDtypeStruct((Q,), jnp.int32),
        mesh=mesh,
        compiler_params=pltpu.CompilerParams(needs_layout_passes=False),
        scratch_types=[
            pltpu.VMEM((QPW * 128,), jnp.float32),
            pltpu.VMEM((SCHUNK,), jnp.float32),
            pltpu.VMEM((SCHUNK,), jnp.float32),
            pltpu.VMEM((SURV_BUF,), jnp.float32),
            pltpu.VMEM((SURV_BUF,), jnp.int32),
            pltpu.VMEM((K_NN + 16,), jnp.int32),
            pltpu.VMEM((A * 32,), jnp.int32),
            pltpu.VMEM((A * 32,), jnp.float32),
            pltpu.VMEM((QPW * 16,), jnp.float32),
            pltpu.VMEM((QPW + 16,), jnp.int32),
            pltpu.SemaphoreType.DMA,
            pltpu.SemaphoreType.DMA,
            pltpu.SemaphoreType.DMA,
        ],
    )(dists, m, qnet16, values_flat)


def kernel(observation, keys, values, W1, b1, W2, b2, W3, b3):
    obs_sq = jnp.sum(observation * observation, axis=-1, keepdims=True)
    key_sq = jnp.sum(keys * keys, axis=-1)
    keys_p = jnp.pad(keys, ((0, CAP_PAD - CAP), (0, 0)))
    key_sq_p = jnp.pad(key_sq, (0, CAP_PAD - CAP), constant_values=np.inf)
    q_net_q = _mlp(observation, W1, b1, W2, b2, W3, b3)
    dists, m = _dists(observation, obs_sq, keys_p, key_sq_p)
    qnet16 = jnp.concatenate([q_net_q, q_net_q], axis=1).reshape(-1)
    return _sc_select(dists, m.reshape(-1), qnet16, values.reshape(-1))
